# baseline (speedup 1.0000x reference)
import functools

import jax
import jax.numpy as jnp
from jax import lax
from jax.experimental import pallas as pl
from jax.experimental.pallas import tpu as pltpu
from jax.experimental.pallas import tpu_sc as plsc

_NC = 2
_NS = 16
_L = 16
_NW = _NC * _NS
_G = 128
_BSC = 16


def _sc_body(num_classes, rpw, yp_hbm, yt_hbm, tci_hbm, out_hbm,
             tci_v, idx_v, yt_v, bufs, acc_v, sem0, sem1):
    c = lax.axis_index("c")
    s = lax.axis_index("s")
    wid = s * _NC + c
    ngrp = rpw // _G
    nch = ngrp * 4
    wpb = 1024 // rpw
    bb = wid // wpb
    off = (wid % wpb) * rpw

    pltpu.sync_copy(tci_hbm.at[bb, pl.ds(off, rpw)], tci_v)
    pltpu.sync_copy(yt_hbm.at[bb, :, pl.ds(off, rpw)], yt_v)

    lanes = lax.iota(jnp.int32, _L)
    sems = [sem0, sem1]
    g0 = off // _G

    def build(i, carry):
        g = i // 8
        i8 = i % 8
        t16 = tci_v[pl.ds(i * _L, _L)]
        base16 = (((bb * num_classes + t16) * 8 + (g0 + g)) * 4) * 8 + i8
        for j in range(4):
            idx_v[g * 4 + j, pl.ds(i8 * _L, _L)] = base16 + j * 8
        return carry

    lax.fori_loop(0, rpw // _L, build, 0)

    def fire(k):
        pltpu.async_copy(yp_hbm.at[idx_v.at[k]], bufs.at[k % 2],
                         sems[k % 2])

    def drain(k):
        pltpu.make_async_copy(yp_hbm.at[idx_v.at[k]], bufs.at[k % 2],
                              sems[k % 2]).wait()

    fire(0)
    lacc = jnp.zeros((_L,), jnp.float32)
    cacc = jnp.zeros((_L,), jnp.float32)
    for k in range(nch):
        if k + 1 < nch:
            fire(k + 1)
        drain(k)
        buf = bufs.at[k % 2]
        g, j = k // 4, k % 4

        def step(i, carry, g=g, j=j, buf=buf):
            la, ca = carry
            r16 = i * _L + lanes
            pb16 = plsc.load_gather(buf, [r16, lanes])
            tb16 = yt_v[j, pl.ds(g * _G + i * _L, _L)]
            t16 = tci_v[pl.ds(g * _G + i * _L, _L)]
            diff = jnp.abs(tb16 - pb16)
            loss = jnp.where(diff < 1.0, 0.5 * diff * diff, diff - 0.5)
            m = t16 > 0
            la = la + jnp.where(m, loss, 0.0)
            ca = ca + jnp.where(m, 1.0, 0.0)
            return la, ca

        lacc, cacc = lax.fori_loop(0, _G // _L, step, (lacc, cacc))

    acc_v[0, :] = lacc
    acc_v[1, :] = cacc
    pltpu.sync_copy(acc_v, out_hbm.at[wid])


def _tc_body(nb, ypT_ref, yt_ref, tci_ref, out_ref, lacc_ref, cacc_ref):
    b = pl.program_id(0)
    cb = pl.program_id(1)

    @pl.when(jnp.logical_and(b == 0, cb == 0))
    def _():
        lacc_ref[...] = jnp.zeros((4, 1024), jnp.float32)
        cacc_ref[...] = jnp.zeros((8, 128), jnp.float32)

    t = tci_ref[0]
    tr = jnp.broadcast_to(t.reshape(1, 1024), (4, 1024))
    ytb = yt_ref[0]

    part = jnp.zeros((4, 1024), jnp.float32)
    for c in range(7):
        cc = cb * 7 + c
        d = jnp.abs(ypT_ref[0, c] - ytb)
        m = jnp.minimum(d, 1.0)
        p = m * (d - 0.5 * m)
        part = part + jnp.where(jnp.logical_and(tr == cc, tr > 0), p, 0.0)
    lacc_ref[...] += part

    @pl.when(cb == 0)
    def _():
        cacc_ref[...] += jnp.where(t > 0, 1.0, 0.0)

    @pl.when(jnp.logical_and(b == nb - 1, cb == 12))
    def _():
        out_ref[0] = lacc_ref[...].reshape(4, 8, 128).sum(axis=0)
        out_ref[1] = cacc_ref[...]


def _reduce_body(parts_hbm, tcp_hbm, out_hbm, parts_v, tcp_v, res_v):
    c = lax.axis_index("c")
    s = lax.axis_index("s")

    @pl.when(jnp.logical_and(s == 0, c == 0))
    def _():
        pltpu.sync_copy(parts_hbm, parts_v)
        pltpu.sync_copy(tcp_hbm, tcp_v)
        zero = jnp.zeros((_L,), jnp.float32)

        def red(i, carry):
            ls, cs = carry
            return ls + parts_v[i, 0, :], cs + parts_v[i, 1, :]

        ls, cs = lax.fori_loop(0, _NW, red, (zero, zero))

        def redtc(i, carry):
            ls, cs = carry
            return (ls + tcp_v[0, i // 8, pl.ds((i % 8) * _L, _L)],
                    cs + tcp_v[1, i // 8, pl.ds((i % 8) * _L, _L)])

        ls2, cs2 = lax.fori_loop(0, 64, redtc, (zero, zero))
        tv = zero + (jnp.sum(ls) + jnp.sum(ls2))
        cv = zero + (jnp.sum(cs) + 4.0 * jnp.sum(cs2))
        mean = tv / jnp.maximum(cv, 1.0)
        res_v[:] = jnp.where(cv > 0.0, mean, zero)
        pltpu.sync_copy(res_v, out_hbm)


@functools.partial(jax.jit, static_argnums=(4, 5))
def _loss(yp16, ypT, yt, tci, num_rows, num_classes):
    rpw = _BSC * 1024 // _NW
    nb = 32 - _BSC
    mesh = plsc.VectorSubcoreMesh(
        core_axis_name="c", subcore_axis_name="s",
        num_cores=_NC, num_subcores=_NS)
    cp = pltpu.CompilerParams(
        needs_layout_passes=False, use_tc_tiling_on_sc=False)
    parts = pl.kernel(
        functools.partial(_sc_body, num_classes, rpw),
        out_type=jax.ShapeDtypeStruct((_NW, 2, _L), jnp.float32),
        mesh=mesh,
        compiler_params=cp,
        scratch_types=[
            pltpu.VMEM((rpw,), jnp.int32),
            pltpu.VMEM((rpw // _G * 4, _G), jnp.int32),
            pltpu.VMEM((4, rpw), jnp.float32),
            pltpu.VMEM((2, _G, _L), jnp.float32),
            pltpu.VMEM((2, _L), jnp.float32),
            pltpu.SemaphoreType.DMA,
            pltpu.SemaphoreType.DMA,
        ],
    )(yp16, yt, tci)
    tcp = pl.pallas_call(
        functools.partial(_tc_body, nb),
        grid=(nb, 13),
        in_specs=[
            pl.BlockSpec((1, 7, 4, 1024), lambda b, cb: (_BSC + b, cb, 0, 0)),
            pl.BlockSpec((1, 4, 1024), lambda b, cb: (_BSC + b, 0, 0)),
            pl.BlockSpec((1, 8, 128), lambda b, cb: (_BSC + b, 0, 0)),
        ],
        out_specs=pl.BlockSpec((2, 8, 128), lambda b, cb: (0, 0, 0)),
        out_shape=jax.ShapeDtypeStruct((2, 8, 128), jnp.float32),
        scratch_shapes=[
            pltpu.VMEM((4, 1024), jnp.float32),
            pltpu.VMEM((8, 128), jnp.float32),
        ],
    )(ypT, yt, tci.reshape(32, 8, 128))
    res = pl.kernel(
        _reduce_body,
        out_type=jax.ShapeDtypeStruct((_L,), jnp.float32),
        mesh=mesh,
        compiler_params=cp,
        scratch_types=[
            pltpu.VMEM((_NW, 2, _L), jnp.float32),
            pltpu.VMEM((2, 8, 128), jnp.float32),
            pltpu.VMEM((_L,), jnp.float32),
        ],
    )(parts, tcp)
    return res[0]


def kernel(y_true, y_pred, target_class_ids):
    B, R, C, _ = y_pred.shape
    N = B * R
    yp16 = (y_pred.reshape(B, R // 128, 128, C, 4)
            .transpose(0, 3, 1, 4, 2)
            .reshape(B * C * (R // 128) * 4 * 8, 16))
    ypT = y_pred.transpose(0, 2, 3, 1)
    yt = y_true.transpose(0, 2, 1)
    return _loss(yp16, ypT, yt, target_class_ids, N, C)


# speedup vs baseline: 2.1843x; 2.1843x over previous
import functools

import jax
import jax.numpy as jnp
from jax import lax
from jax.experimental import pallas as pl
from jax.experimental.pallas import tpu as pltpu
from jax.experimental.pallas import tpu_sc as plsc

_NC = 2
_NS = 16
_L = 16
_NW = _NC * _NS
_G = 128


def _partials_body(num_classes, rpw, b_off, yp_hbm, yt_hbm, tci_hbm, out_hbm,
                   tci_v, idx_v, yt_v, bufs, acc_v, sem0, sem1):
    s = lax.axis_index("s")
    wid = s
    ngrp = rpw // _G
    nch = ngrp * 4

    bb = b_off + wid
    pltpu.sync_copy(tci_hbm.at[bb], tci_v)
    pltpu.sync_copy(yt_hbm.at[bb], yt_v)

    lanes = lax.iota(jnp.int32, _L)
    sems = [sem0, sem1]

    def build(i, carry):
        g = i // 8
        i8 = i % 8
        t16 = tci_v[pl.ds(i * _L, _L)]
        base16 = (((bb * num_classes + t16) * ngrp + g) * 4) * 8 + i8
        for j in range(4):
            idx_v[g * 4 + j, pl.ds(i8 * _L, _L)] = base16 + j * 8
        return carry

    lax.fori_loop(0, rpw // _L, build, 0)

    def fire(k):
        pltpu.async_copy(yp_hbm.at[idx_v.at[k]], bufs.at[k % 2],
                         sems[k % 2])

    def drain(k):
        pltpu.make_async_copy(yp_hbm.at[idx_v.at[k]], bufs.at[k % 2],
                              sems[k % 2]).wait()

    fire(0)
    lacc = jnp.zeros((_L,), jnp.float32)
    cacc = jnp.zeros((_L,), jnp.float32)
    for k in range(nch):
        if k + 1 < nch:
            fire(k + 1)
        drain(k)
        buf = bufs.at[k % 2]
        g, j = k // 4, k % 4

        def step(i, carry, g=g, j=j, buf=buf):
            la, ca = carry
            r16 = i * _L + lanes
            pb16 = plsc.load_gather(buf, [r16, lanes])
            tb16 = yt_v[j, pl.ds(g * _G + i * _L, _L)]
            t16 = tci_v[pl.ds(g * _G + i * _L, _L)]
            diff = jnp.abs(tb16 - pb16)
            loss = jnp.where(diff < 1.0, 0.5 * diff * diff, diff - 0.5)
            m = t16 > 0
            la = la + jnp.where(m, loss, 0.0)
            ca = ca + jnp.where(m, 1.0, 0.0)
            return la, ca

        lacc, cacc = lax.fori_loop(0, _G // _L, step, (lacc, cacc))

    acc_v[0, :] = lacc
    acc_v[1, :] = cacc
    pltpu.sync_copy(acc_v, out_hbm.at[wid])


def _reduce_body(pa_hbm, pb_hbm, out_hbm, pa_v, pb_v, res_v):
    s = lax.axis_index("s")

    @pl.when(s == 0)
    def _():
        pltpu.sync_copy(pa_hbm, pa_v)
        pltpu.sync_copy(pb_hbm, pb_v)
        zero = jnp.zeros((_L,), jnp.float32)

        def red(i, carry):
            ls, cs = carry
            return (ls + pa_v[i, 0, :] + pb_v[i, 0, :],
                    cs + pa_v[i, 1, :] + pb_v[i, 1, :])

        ls, cs = lax.fori_loop(0, _NS, red, (zero, zero))
        tv = zero + jnp.sum(ls)
        cv = zero + jnp.sum(cs)
        mean = tv / jnp.maximum(cv, 1.0)
        res_v[:] = jnp.where(cv > 0.0, mean, zero)
        pltpu.sync_copy(res_v, out_hbm)


@functools.partial(jax.jit, static_argnums=(3, 4))
def _sc_loss(yp, yt, tci, num_rows, num_classes):
    rpw = 1024
    mesh1 = plsc.VectorSubcoreMesh(
        core_axis_name="c", subcore_axis_name="s",
        num_cores=1, num_subcores=_NS)
    cp = pltpu.CompilerParams(
        needs_layout_passes=False, use_tc_tiling_on_sc=False)

    def half(b_off):
        return pl.kernel(
            functools.partial(_partials_body, num_classes, rpw, b_off),
            out_type=jax.ShapeDtypeStruct((_NS, 2, _L), jnp.float32),
            mesh=mesh1,
            compiler_params=cp,
            scratch_types=[
                pltpu.VMEM((rpw,), jnp.int32),
                pltpu.VMEM((rpw // _G * 4, _G), jnp.int32),
                pltpu.VMEM((4, rpw), jnp.float32),
                pltpu.VMEM((2, _G, _L), jnp.float32),
                pltpu.VMEM((2, _L), jnp.float32),
                pltpu.SemaphoreType.DMA,
                pltpu.SemaphoreType.DMA,
            ],
        )(yp, yt, tci)

    parts_a = half(0)
    parts_b = half(_NS)
    res = pl.kernel(
        _reduce_body,
        out_type=jax.ShapeDtypeStruct((_L,), jnp.float32),
        mesh=mesh1,
        compiler_params=cp,
        scratch_types=[
            pltpu.VMEM((_NS, 2, _L), jnp.float32),
            pltpu.VMEM((_NS, 2, _L), jnp.float32),
            pltpu.VMEM((_L,), jnp.float32),
        ],
    )(parts_a, parts_b)
    return res[0]


def kernel(y_true, y_pred, target_class_ids):
    B, R, C, _ = y_pred.shape
    N = B * R
    yp = (y_pred.reshape(B, R // 128, 128, C, 4)
          .transpose(0, 3, 1, 4, 2)
          .reshape(B * C * (R // 128) * 4 * 8, 16))
    yt = y_true.transpose(0, 2, 1)
    return _sc_loss(yp, yt, target_class_ids, N, C)


# speedup vs baseline: 3.1988x; 1.4644x over previous
import functools

import jax
import jax.numpy as jnp
from jax import lax
from jax.experimental import pallas as pl
from jax.experimental.pallas import tpu as pltpu
from jax.experimental.pallas import tpu_sc as plsc

_NC = 2
_NS = 16
_L = 16
_NW = _NC * _NS
_G = 128


def _partials_body(num_classes, rpw, yp_hbm, yt_hbm, tci_hbm, out_hbm,
                   tci_v, idx_v, yt_v, bufs, acc_v, sem0, sem1):
    c = lax.axis_index("c")
    s = lax.axis_index("s")
    wid = s * _NC + c
    ngrp = rpw // _G
    nch = ngrp * 4

    pltpu.sync_copy(tci_hbm.at[wid], tci_v)
    pltpu.sync_copy(yt_hbm.at[wid], yt_v)

    lanes = lax.iota(jnp.int32, _L)
    sems = [sem0, sem1]

    def build(i, carry):
        g = i // 8
        i8 = i % 8
        t16 = tci_v[pl.ds(i * _L, _L)]
        base16 = (((wid * num_classes + t16) * ngrp + g) * 4) * 8 + i8
        for j in range(4):
            idx_v[g * 4 + j, pl.ds(i8 * _L, _L)] = base16 + j * 8
        return carry

    lax.fori_loop(0, rpw // _L, build, 0)

    def fire(k):
        pltpu.async_copy(yp_hbm.at[idx_v.at[k]], bufs.at[k % 2],
                         sems[k % 2])

    def drain(k):
        pltpu.make_async_copy(yp_hbm.at[idx_v.at[k]], bufs.at[k % 2],
                              sems[k % 2]).wait()

    fire(0)
    lacc = jnp.zeros((_L,), jnp.float32)
    cacc = jnp.zeros((_L,), jnp.float32)
    for k in range(nch):
        if k + 1 < nch:
            fire(k + 1)
        drain(k)
        buf = bufs.at[k % 2]
        g, j = k // 4, k % 4

        def step(i, carry, g=g, j=j, buf=buf):
            la, ca = carry
            r16 = i * _L + lanes
            pb16 = plsc.load_gather(buf, [r16, lanes])
            tb16 = yt_v[j, pl.ds(g * _G + i * _L, _L)]
            t16 = tci_v[pl.ds(g * _G + i * _L, _L)]
            diff = jnp.abs(tb16 - pb16)
            loss = jnp.where(diff < 1.0, 0.5 * diff * diff, diff - 0.5)
            m = t16 > 0
            la = la + jnp.where(m, loss, 0.0)
            ca = ca + jnp.where(m, 1.0, 0.0)
            return la, ca

        lacc, cacc = lax.fori_loop(0, _G // _L, step, (lacc, cacc))

    acc_v[0, :] = lacc
    acc_v[1, :] = cacc
    pltpu.sync_copy(acc_v, out_hbm.at[wid])


def _reduce_body(parts_hbm, out_hbm, parts_v, res_v):
    c = lax.axis_index("c")
    s = lax.axis_index("s")

    @pl.when(jnp.logical_and(s == 0, c == 0))
    def _():
        pltpu.sync_copy(parts_hbm, parts_v)
        zero = jnp.zeros((_L,), jnp.float32)

        def red(i, carry):
            ls, cs = carry
            return ls + parts_v[i, 0, :], cs + parts_v[i, 1, :]

        ls, cs = lax.fori_loop(0, _NW, red, (zero, zero))
        tv = zero + jnp.sum(ls)
        cv = zero + jnp.sum(cs)
        mean = tv / jnp.maximum(cv, 1.0)
        res_v[:] = jnp.where(cv > 0.0, mean, zero)
        pltpu.sync_copy(res_v, out_hbm)


@functools.partial(jax.jit, static_argnums=(3, 4))
def _sc_loss(yp, yt, tci, num_rows, num_classes):
    rpw = num_rows // _NW
    mesh = plsc.VectorSubcoreMesh(
        core_axis_name="c", subcore_axis_name="s",
        num_cores=_NC, num_subcores=_NS)
    cp = pltpu.CompilerParams(
        needs_layout_passes=False, use_tc_tiling_on_sc=False)
    parts = pl.kernel(
        functools.partial(_partials_body, num_classes, rpw),
        out_type=jax.ShapeDtypeStruct((_NW, 2, _L), jnp.float32),
        mesh=mesh,
        compiler_params=cp,
        scratch_types=[
            pltpu.VMEM((rpw,), jnp.int32),
            pltpu.VMEM((rpw // _G * 4, _G), jnp.int32),
            pltpu.VMEM((4, rpw), jnp.float32),
            pltpu.VMEM((2, _G, _L), jnp.float32),
            pltpu.VMEM((2, _L), jnp.float32),
            pltpu.SemaphoreType.DMA,
            pltpu.SemaphoreType.DMA,
        ],
    )(yp, yt, tci)
    res = pl.kernel(
        _reduce_body,
        out_type=jax.ShapeDtypeStruct((_L,), jnp.float32),
        mesh=mesh,
        compiler_params=cp,
        scratch_types=[
            pltpu.VMEM((_NW, 2, _L), jnp.float32),
            pltpu.VMEM((_L,), jnp.float32),
        ],
    )(parts)
    return res[0]


def kernel(y_true, y_pred, target_class_ids):
    B, R, C, _ = y_pred.shape
    N = B * R
    yp = (y_pred.reshape(B, R // 128, 128, C, 4)
          .transpose(0, 3, 1, 4, 2)
          .reshape(B * C * (R // 128) * 4 * 8, 16))
    yt = y_true.transpose(0, 2, 1)
    return _sc_loss(yp, yt, target_class_ids, N, C)
